# baseline (speedup 1.0000x reference)
import jax
import jax.numpy as jnp
import numpy as np
from jax.experimental import pallas as pl
from jax.experimental.pallas import tpu as pltpu


def _conv_tiles(x4, rhs_ref, t, H, D, WT, KW):
    C = 4
    xw = x4[:, :, (WT * C) * t: (WT * C) * t + KW * C]
    R = jnp.concatenate([xw[:, kh:kh + H, :] for kh in range(3)],
                        axis=2)
    acc = None
    for kd in range(3):
        lhs = R[kd:kd + D].reshape(D * H, 3 * KW * C)
        p = jnp.dot(lhs, rhs_ref[kd], preferred_element_type=jnp.float32)
        acc = p if acc is None else acc + p
    return acc


def _make_prep_stats_kernel(C, D, H, W, WT, KW, NT, NB):
    def _body(x_ref, pc_ref, rhs_ref, xp_ref, sum_ref, sq_ref):
        xp_ref[...] = jnp.zeros_like(xp_ref)
        for i in range(NB):
            xr = x_ref[i].reshape(C, D * H, W).astype(jnp.bfloat16)
            acc = None
            for c in range(C):
                p = jnp.dot(xr[c], pc_ref[c],
                            preferred_element_type=jnp.float32)
                acc = p if acc is None else acc + p
            xp_ref[i, 1:D + 1, 1:H + 1, C:C * (W + 1)] = (
                acc.astype(jnp.bfloat16).reshape(D, H, W * C))
            x4 = xp_ref[i]
            s = jnp.zeros((1, sum_ref.shape[-1]), jnp.float32)
            q = jnp.zeros_like(s)
            for t in range(NT):
                a = _conv_tiles(x4, rhs_ref, t, H, D, WT, KW)
                s = s + jnp.sum(a, axis=0, keepdims=True)
                q = q + jnp.sum(a * a, axis=0, keepdims=True)
            sum_ref[i] = s
            sq_ref[i] = q
    return _body


def _make_apply_kernel(D, H, W, WT, KW, NT, Cout, r, NB):
    Dp = D // (r * r)

    def _body(x_ref, rhs_ref, scale_ref, shift_ref, pe_ref, po_ref, out_ref):
        for i in range(NB):
            x4 = x_ref[i]
            ys = []
            for t in range(NT):
                acc = _conv_tiles(x4, rhs_ref, t, H, D, WT, KW)
                y = jnp.maximum(acc * scale_ref[...] + shift_ref[...], 0.0)
                ys.append(y.astype(jnp.bfloat16))
            yy = jnp.concatenate(ys, axis=1)
            y5 = yy.reshape(Dp, r, r, H, W * Cout).transpose(0, 3, 1, 2, 4)
            ye = y5[:, :, :, 0].reshape(Dp * r * H, W * Cout)
            yo = y5[:, :, :, 1].reshape(Dp * r * H, W * Cout)
            z = (jnp.dot(ye, pe_ref[...], preferred_element_type=jnp.float32) +
                 jnp.dot(yo, po_ref[...], preferred_element_type=jnp.float32))
            g = z.reshape(Dp, r * H, r * W * Cout)
            for co in range(Cout):
                out_ref[i, co] = g[:, :, r * W * co: r * W * (co + 1)]
    return _body


def _ecre_opt(x, w, gamma, beta, up_scale=2, eps=1e-5):
    N, C, D, H, W = x.shape
    Cout = int(w.shape[0])
    r = up_scale
    Dp = D // (r * r)
    WT = 16
    KW = WT + 2
    NT = W // WT
    K = 3 * KW * C
    NL = WT * Cout
    WCo = W * Cout

    pc_np = np.zeros((C, W, W * C), np.float32)
    for c in range(C):
        pc_np[c, np.arange(W), np.arange(W) * C + c] = 1.0
    pc = jnp.asarray(pc_np, dtype=jnp.bfloat16)

    w_t = jnp.transpose(w, (2, 3, 4, 1, 0)).astype(jnp.float32)
    scat = np.zeros((3, WT, KW), np.float32)
    for kw in range(3):
        scat[kw, np.arange(WT), np.arange(WT) + kw] = 1.0
    rhs = jnp.einsum('dhkcq,kpr->dhrcpq', w_t, scat).reshape(3, K, NL)
    rhs = rhs.astype(jnp.bfloat16)

    NB = 2
    x_spec = pl.BlockSpec((1, D + 2, H + 2, (W + 2) * C), lambda n: (n, 0, 0, 0))
    w_spec = pl.BlockSpec((3, K, NL), lambda n: (0, 0, 0))

    xp, sum_part, sq_part = pl.pallas_call(
        _make_prep_stats_kernel(C, D, H, W, WT, KW, NT, NB),
        out_shape=(jax.ShapeDtypeStruct((N, D + 2, H + 2, (W + 2) * C),
                                        jnp.bfloat16),
                   jax.ShapeDtypeStruct((N, 1, NL), jnp.float32),
                   jax.ShapeDtypeStruct((N, 1, NL), jnp.float32)),
        grid=(N // NB,),
        in_specs=[pl.BlockSpec((NB, C, D, H, W), lambda n: (n, 0, 0, 0, 0)),
                  pl.BlockSpec((C, W, W * C), lambda n: (0, 0, 0)),
                  w_spec],
        out_specs=(pl.BlockSpec((NB, D + 2, H + 2, (W + 2) * C),
                                lambda n: (n, 0, 0, 0)),
                   pl.BlockSpec((NB, 1, NL), lambda n: (n, 0, 0)),
                   pl.BlockSpec((NB, 1, NL), lambda n: (n, 0, 0))),
        compiler_params=pltpu.CompilerParams(
            dimension_semantics=("parallel",)),
    )(x, pc, rhs)

    cnt = float(N * D * H * W)
    s_c = jnp.sum(sum_part, axis=(0, 1)).reshape(WT, Cout).sum(axis=0)
    q_c = jnp.sum(sq_part, axis=(0, 1)).reshape(WT, Cout).sum(axis=0)
    mean = s_c / cnt
    var = jnp.maximum(q_c / cnt - mean * mean, 0.0)
    inv = gamma.astype(jnp.float32) * jax.lax.rsqrt(var + eps)
    scale_row = jnp.tile(inv, WT).reshape(1, NL)
    shift_row = jnp.tile(beta.astype(jnp.float32) - mean * inv,
                         WT).reshape(1, NL)

    pe_np = np.zeros((WCo, r * WCo), np.float32)
    po_np = np.zeros((WCo, r * WCo), np.float32)
    for co in range(Cout):
        for wg in range(W):
            pe_np[wg * Cout + co, co * (r * W) + 2 * wg] = 1.0
            po_np[wg * Cout + co, co * (r * W) + 2 * wg + 1] = 1.0
    pe = jnp.asarray(pe_np, dtype=jnp.bfloat16)
    po = jnp.asarray(po_np, dtype=jnp.bfloat16)

    out = pl.pallas_call(
        _make_apply_kernel(D, H, W, WT, KW, NT, Cout, r, NB),
        out_shape=jax.ShapeDtypeStruct((N, Cout, Dp, r * H, r * W), jnp.float32),
        grid=(N // NB,),
        in_specs=[pl.BlockSpec((NB, D + 2, H + 2, (W + 2) * C),
                               lambda n: (n, 0, 0, 0)),
                  w_spec,
                  pl.BlockSpec((1, NL), lambda n: (0, 0)),
                  pl.BlockSpec((1, NL), lambda n: (0, 0)),
                  pl.BlockSpec((WCo, r * WCo), lambda n: (0, 0)),
                  pl.BlockSpec((WCo, r * WCo), lambda n: (0, 0))],
        out_specs=pl.BlockSpec((NB, Cout, Dp, r * H, r * W),
                               lambda n: (n, 0, 0, 0, 0)),
        compiler_params=pltpu.CompilerParams(
            dimension_semantics=("parallel",)),
    )(xp, rhs, scale_row, shift_row, pe, po)
    return out


def kernel(x, w, b, gamma, beta):
    del b
    return _ecre_opt(x, w, gamma, beta, up_scale=2)


# speedup vs baseline: 1.1842x; 1.1842x over previous
import jax
import jax.numpy as jnp
import numpy as np
from jax.experimental import pallas as pl
from jax.experimental.pallas import tpu as pltpu


def _conv_tiles(x4, rhs_ref, t, H, D, WT, KW):
    C = 4
    xw = x4[:, :, (WT * C) * t: (WT * C) * t + KW * C]
    R = jnp.concatenate([xw[:, kh:kh + H, :] for kh in range(3)],
                        axis=2)
    acc = None
    for kd in range(3):
        lhs = R[kd:kd + D].reshape(D * H, 3 * KW * C)
        p = jnp.dot(lhs, rhs_ref[kd], preferred_element_type=jnp.float32)
        acc = p if acc is None else acc + p
    return acc


def _make_prep_stats_kernel(C, D, H, W, WT, KW, NT, NB):
    def _body(x_ref, pc_ref, rhs_ref, xp_ref, sum_ref, sq_ref):
        xp_ref[...] = jnp.zeros_like(xp_ref)
        for i in range(NB):
            xr = x_ref[i].reshape(C, D * H, W).astype(jnp.bfloat16)
            acc = None
            for c in range(C):
                p = jnp.dot(xr[c], pc_ref[c],
                            preferred_element_type=jnp.float32)
                acc = p if acc is None else acc + p
            xp_ref[i, 1:D + 1, 1:H + 1, C:C * (W + 1)] = (
                acc.astype(jnp.bfloat16).reshape(D, H, W * C))
            x4 = xp_ref[i]
            s = jnp.zeros((1, sum_ref.shape[-1]), jnp.float32)
            q = jnp.zeros_like(s)
            for t in range(NT):
                a = _conv_tiles(x4, rhs_ref, t, H, D, WT, KW)
                s = s + jnp.sum(a, axis=0, keepdims=True)
                q = q + jnp.sum(a * a, axis=0, keepdims=True)
            sum_ref[i] = s
            sq_ref[i] = q
    return _body


def _make_apply_kernel(D, H, W, WT, KW, NT, Cout, r, NB):
    Dp = D // (r * r)

    def _body(x_ref, rhs_ref, scale_ref, shift_ref, pe_ref, po_ref, out_ref):
        for i in range(NB):
            x4 = x_ref[i]
            ys = []
            for t in range(NT):
                acc = _conv_tiles(x4, rhs_ref, t, H, D, WT, KW)
                y = jnp.maximum(acc * scale_ref[...] + shift_ref[...], 0.0)
                ys.append(y.astype(jnp.bfloat16))
            yy = jnp.concatenate(ys, axis=1)
            y4 = yy.reshape(Dp * r, r, H, W * Cout)
            ye = y4[:, 0].reshape(Dp * r * H, W * Cout)
            yo = y4[:, 1].reshape(Dp * r * H, W * Cout)
            CH = Cout // 2
            HL = r * W * CH
            for half in range(2):
                pe_h = pe_ref[:, HL * half: HL * (half + 1)]
                po_h = po_ref[:, HL * half: HL * (half + 1)]
                z = (jnp.dot(ye, pe_h, preferred_element_type=jnp.float32) +
                     jnp.dot(yo, po_h, preferred_element_type=jnp.float32))
                g = (z.reshape(Dp, r, H, HL)
                     .transpose(0, 2, 1, 3)
                     .reshape(Dp, r * H, HL))
                for co in range(CH):
                    out_ref[i, CH * half + co] = (
                        g[:, :, r * W * co: r * W * (co + 1)])
    return _body


def _ecre_opt(x, w, gamma, beta, up_scale=2, eps=1e-5):
    N, C, D, H, W = x.shape
    Cout = int(w.shape[0])
    r = up_scale
    Dp = D // (r * r)
    WT = 16
    KW = WT + 2
    NT = W // WT
    K = 3 * KW * C
    NL = WT * Cout
    WCo = W * Cout

    pc_np = np.zeros((C, W, W * C), np.float32)
    for c in range(C):
        pc_np[c, np.arange(W), np.arange(W) * C + c] = 1.0
    pc = jnp.asarray(pc_np, dtype=jnp.bfloat16)

    w_t = jnp.transpose(w, (2, 3, 4, 1, 0)).astype(jnp.float32)
    scat = np.zeros((3, WT, KW), np.float32)
    for kw in range(3):
        scat[kw, np.arange(WT), np.arange(WT) + kw] = 1.0
    rhs = jnp.einsum('dhkcq,kpr->dhrcpq', w_t, scat).reshape(3, K, NL)
    rhs = rhs.astype(jnp.bfloat16)

    NB = 2
    x_spec = pl.BlockSpec((1, D + 2, H + 2, (W + 2) * C), lambda n: (n, 0, 0, 0))
    w_spec = pl.BlockSpec((3, K, NL), lambda n: (0, 0, 0))

    xp, sum_part, sq_part = pl.pallas_call(
        _make_prep_stats_kernel(C, D, H, W, WT, KW, NT, NB),
        out_shape=(jax.ShapeDtypeStruct((N, D + 2, H + 2, (W + 2) * C),
                                        jnp.bfloat16),
                   jax.ShapeDtypeStruct((N, 1, NL), jnp.float32),
                   jax.ShapeDtypeStruct((N, 1, NL), jnp.float32)),
        grid=(N // NB,),
        in_specs=[pl.BlockSpec((NB, C, D, H, W), lambda n: (n, 0, 0, 0, 0)),
                  pl.BlockSpec((C, W, W * C), lambda n: (0, 0, 0)),
                  w_spec],
        out_specs=(pl.BlockSpec((NB, D + 2, H + 2, (W + 2) * C),
                                lambda n: (n, 0, 0, 0)),
                   pl.BlockSpec((NB, 1, NL), lambda n: (n, 0, 0)),
                   pl.BlockSpec((NB, 1, NL), lambda n: (n, 0, 0))),
        compiler_params=pltpu.CompilerParams(
            dimension_semantics=("parallel",)),
    )(x, pc, rhs)

    cnt = float(N * D * H * W)
    s_c = jnp.sum(sum_part, axis=(0, 1)).reshape(WT, Cout).sum(axis=0)
    q_c = jnp.sum(sq_part, axis=(0, 1)).reshape(WT, Cout).sum(axis=0)
    mean = s_c / cnt
    var = jnp.maximum(q_c / cnt - mean * mean, 0.0)
    inv = gamma.astype(jnp.float32) * jax.lax.rsqrt(var + eps)
    scale_row = jnp.tile(inv, WT).reshape(1, NL)
    shift_row = jnp.tile(beta.astype(jnp.float32) - mean * inv,
                         WT).reshape(1, NL)

    pe_np = np.zeros((WCo, r * WCo), np.float32)
    po_np = np.zeros((WCo, r * WCo), np.float32)
    for co in range(Cout):
        for wg in range(W):
            pe_np[wg * Cout + co, co * (r * W) + 2 * wg] = 1.0
            po_np[wg * Cout + co, co * (r * W) + 2 * wg + 1] = 1.0
    pe = jnp.asarray(pe_np, dtype=jnp.bfloat16)
    po = jnp.asarray(po_np, dtype=jnp.bfloat16)

    out = pl.pallas_call(
        _make_apply_kernel(D, H, W, WT, KW, NT, Cout, r, NB),
        out_shape=jax.ShapeDtypeStruct((N, Cout, Dp, r * H, r * W), jnp.float32),
        grid=(N // NB,),
        in_specs=[pl.BlockSpec((NB, D + 2, H + 2, (W + 2) * C),
                               lambda n: (n, 0, 0, 0)),
                  w_spec,
                  pl.BlockSpec((1, NL), lambda n: (0, 0)),
                  pl.BlockSpec((1, NL), lambda n: (0, 0)),
                  pl.BlockSpec((WCo, r * WCo), lambda n: (0, 0)),
                  pl.BlockSpec((WCo, r * WCo), lambda n: (0, 0))],
        out_specs=pl.BlockSpec((NB, Cout, Dp, r * H, r * W),
                               lambda n: (n, 0, 0, 0, 0)),
        compiler_params=pltpu.CompilerParams(
            dimension_semantics=("parallel",)),
    )(xp, rhs, scale_row, shift_row, pe, po)
    return out


def kernel(x, w, b, gamma, beta):
    del b
    return _ecre_opt(x, w, gamma, beta, up_scale=2)
